# baseline (speedup 1.0000x reference)
import functools

import jax
import jax.numpy as jnp
from jax import lax
from jax.experimental import pallas as pl
from jax.experimental.pallas import tpu as pltpu
from jax.experimental.pallas import tpu_sc as plsc

_L = 16
_W = 128
_B = 64


def _fill_f32(ref, rows, cols, value):
    nv = cols // _L

    def body(k, carry):
        i = k // nv
        j = (k % nv) * _L
        ref[i, pl.ds(j, _L)] = jnp.full((_L,), value, jnp.float32)
        return carry

    lax.fori_loop(0, rows * nv, body, 0)


def _zero_shared(zbuf, acc, s, rows_per_tile, zrows):
    base = s * rows_per_tile
    off = 0
    while off < rows_per_tile:
        nr = min(zrows, rows_per_tile - off)
        pltpu.sync_copy(zbuf.at[pl.ds(0, nr)], acc.at[pl.ds(base + off, nr)])
        off += nr


def _copy_out(acc, out_ref, s, n_rows):
    rpt = (n_rows // 16) // 8 * 8
    pltpu.sync_copy(acc.at[pl.ds(s * rpt, rpt)], out_ref.at[pl.ds(s * rpt, rpt)])
    rem = n_rows - rpt * 16
    if rem:
        @pl.when(s == 15)
        def _():
            pltpu.sync_copy(acc.at[pl.ds(rpt * 16, rem)],
                            out_ref.at[pl.ds(rpt * 16, rem)])


def _acc_rows(n):
    return (n + 16 + 127) // 128 * 128


def _make_seg_sum(n, ep, nway):
    np_acc = _acc_rows(n)
    nb = ep // (nway * _B)
    nbc = min(nb, 80)
    assert nb % nbc == 0 and nbc % 2 == 0
    rows_per_tile_z = np_acc // 16
    mesh = plsc.VectorSubcoreMesh(core_axis_name="c", subcore_axis_name="s")

    @functools.partial(
        pl.kernel,
        out_type=(jax.ShapeDtypeStruct((n, _W), jnp.float32),
                  jax.ShapeDtypeStruct((n, _W), jnp.float32)),
        mesh=mesh,
        scratch_types=dict(
            idx_s=pltpu.VMEM((nbc, _B), jnp.int32),
            idx_d=pltpu.VMEM((nbc, _B), jnp.int32),
            rows_a=pltpu.VMEM((_B, _W), jnp.float32),
            rows_b=pltpu.VMEM((_B, _W), jnp.float32),
            sem_a=pltpu.SemaphoreType.DMA,
            sem_b=pltpu.SemaphoreType.DMA,
            sem_sa=pltpu.SemaphoreType.DMA,
            sem_sb=pltpu.SemaphoreType.DMA,
            acc=pltpu.VMEM_SHARED((np_acc, _W), jnp.float32),
        ),
    )
    def seg_sum(x0, x1, src3d, dst3d, out0, out1,
                idx_s, idx_d, rows_a, rows_b, sem_a, sem_b, sem_sa, sem_sb,
                acc):
        c = lax.axis_index("c")
        s = lax.axis_index("s")
        _fill_f32(rows_a, _B, _W, 0.0)
        _zero_shared(rows_a, acc, s, rows_per_tile_z, _B)
        slab = s if nway == 16 else s * 2 + c
        plsc.subcore_barrier()

        def run(x_ref, out_ref):
            def fire(rows, b, sem):
                pltpu.async_copy(x_ref.at[idx_s.at[b]], rows, sem)

            def drain(rows, b, sem):
                pltpu.make_async_copy(x_ref.at[idx_s.at[b]], rows, sem).wait()

            def scat_fire(rows, b, sem):
                pltpu.async_copy(rows, acc.at[idx_d.at[b]], sem, add=True)

            def scat_drain(rows, b, sem):
                pltpu.make_async_copy(rows, acc.at[idx_d.at[b]], sem).wait()

            def chunk(ci, carry):
                r0 = pl.multiple_of(ci * nbc, 8)
                pltpu.sync_copy(src3d.at[slab, pl.ds(r0, nbc)], idx_s)
                pltpu.sync_copy(dst3d.at[slab, pl.ds(r0, nbc)], idx_d)
                fire(rows_a, 0, sem_a)

                def body(q, carry2):
                    b0 = 2 * q
                    fire(rows_b, b0 + 1, sem_b)
                    drain(rows_a, b0, sem_a)
                    scat_fire(rows_a, b0, sem_sa)
                    drain(rows_b, b0 + 1, sem_b)
                    scat_fire(rows_b, b0 + 1, sem_sb)
                    scat_drain(rows_a, b0, sem_sa)

                    @pl.when(b0 + 2 < nbc)
                    def _():
                        fire(rows_a, b0 + 2, sem_a)

                    scat_drain(rows_b, b0 + 1, sem_sb)
                    return carry2

                lax.fori_loop(0, nbc // 2, body, 0)
                return carry

            lax.fori_loop(0, nb // nbc, chunk, 0)
            plsc.subcore_barrier()
            _copy_out(acc, out_ref, s, n)

        @pl.when(c == 0)
        def _():
            run(x0, out0)

        @pl.when(c == 1)
        def _():
            run(x1, out1)

    return seg_sum


def _make_deg(n, ep):
    np_acc = _acc_rows(n)
    nb = ep // (32 * 128)
    rows_per_tile_z = np_acc // 16
    mesh = plsc.VectorSubcoreMesh(core_axis_name="c", subcore_axis_name="s")

    @functools.partial(
        pl.kernel,
        out_type=(jax.ShapeDtypeStruct((n, _W), jnp.float32),
                  jax.ShapeDtypeStruct((n, _W), jnp.float32)),
        mesh=mesh,
        scratch_types=dict(
            ones_r=pltpu.VMEM((128, _W), jnp.float32),
            idxb=pltpu.VMEM((nb, 128), jnp.int32),
            dacc=pltpu.VMEM_SHARED((np_acc, _W), jnp.float32),
        ),
    )
    def deg_kernel(dst3d, deg0, deg1, ones_r, idxb, dacc):
        c = lax.axis_index("c")
        s = lax.axis_index("s")
        _fill_f32(ones_r, 128, _W, 0.0)
        _zero_shared(ones_r, dacc, s, rows_per_tile_z, 128)
        _fill_f32(ones_r, 128, _W, 1.0)
        w = s * 2 + c
        pltpu.sync_copy(dst3d.at[w], idxb)
        plsc.subcore_barrier()

        def body(b, carry):
            pltpu.sync_copy(ones_r, dacc.at[idxb.at[b]], add=True)
            return carry

        lax.fori_loop(0, nb, body, 0)
        plsc.subcore_barrier()

        @pl.when(c == 0)
        def _():
            _copy_out(dacc, deg0, s, n)

        @pl.when(c == 1)
        def _():
            _copy_out(dacc, deg1, s, n)

    return deg_kernel


_R = 512


def _dinv(d0, d1):
    return lax.rsqrt(d0[:, :1] + d1[:, :1] + 1.0)


def _blk(i):
    return (i, 0)


def _fix(i):
    return (0, 0)


def _tc_l1(xa, xb, w, b, d0, d1, inv_n):
    n = xa.shape[0]
    dmid = w.shape[1]

    def body(xa_ref, xb_ref, w_ref, b_ref, d0_ref, d1_ref, o_ref):
        h = jnp.dot(xa_ref[...] + xb_ref[...], w_ref[...],
                    preferred_element_type=jnp.float32)
        h = jnp.maximum(h * inv_n + b_ref[...], 0.0)
        o_ref[...] = h * _dinv(d0_ref[...], d1_ref[...])

    return pl.pallas_call(
        body,
        grid=(pl.cdiv(n, _R),),
        in_specs=[
            pl.BlockSpec((_R, _W), _blk),
            pl.BlockSpec((_R, _W), _blk),
            pl.BlockSpec((_W, dmid), _fix),
            pl.BlockSpec((1, dmid), _fix),
            pl.BlockSpec((_R, _W), _blk),
            pl.BlockSpec((_R, _W), _blk),
        ],
        out_specs=pl.BlockSpec((_R, _W), _blk),
        out_shape=jax.ShapeDtypeStruct((n, _W), jnp.float32),
    )(xa, xb, w, b, d0, d1)


def _tc_layer(xa, xb, wa, wb, b, wg, d0, d1, inv_n):
    n = xa.shape[0]
    dmid = wa.shape[1]
    dout = wg.shape[1]
    nout = (dout + _W - 1) // _W

    def body(xa_ref, xb_ref, wa_ref, wb_ref, b_ref, wg_ref, d0_ref, d1_ref,
             *o_refs):
        h = (jnp.dot(xa_ref[...], wa_ref[...], preferred_element_type=jnp.float32)
             + jnp.dot(xb_ref[...], wb_ref[...], preferred_element_type=jnp.float32))
        h = jnp.maximum(h * inv_n + b_ref[...], 0.0)
        y = jnp.dot(h, wg_ref[...], preferred_element_type=jnp.float32)
        y = y * _dinv(d0_ref[...], d1_ref[...])
        for k in range(nout):
            lo = k * _W
            wk = min(_W, dout - lo)
            blk = y[:, lo:lo + wk]
            if wk < _W:
                blk = jnp.concatenate(
                    [blk, jnp.zeros((blk.shape[0], _W - wk), jnp.float32)],
                    axis=1)
            o_refs[k][...] = blk

    return pl.pallas_call(
        body,
        grid=(pl.cdiv(n, _R),),
        in_specs=[
            pl.BlockSpec((_R, _W), _blk),
            pl.BlockSpec((_R, _W), _blk),
            pl.BlockSpec((_W, dmid), _fix),
            pl.BlockSpec((_W, dmid), _fix),
            pl.BlockSpec((1, dmid), _fix),
            pl.BlockSpec((dmid, dout), _fix),
            pl.BlockSpec((_R, _W), _blk),
            pl.BlockSpec((_R, _W), _blk),
        ],
        out_specs=[pl.BlockSpec((_R, _W), _blk)] * nout,
        out_shape=[jax.ShapeDtypeStruct((n, _W), jnp.float32)] * nout,
    )(xa, xb, wa, wb, b, wg, d0, d1)


def _tc_mid(za, zb, hd, wg, bg, d0, d1):
    n = za.shape[0]
    dmid = wg.shape[1]

    def body(za_ref, zb_ref, hd_ref, wg_ref, bg_ref, d0_ref, d1_ref,
             ol_ref, oh_ref):
        u = jnp.dot(za_ref[...] + zb_ref[...] + hd_ref[...], wg_ref[...],
                    preferred_element_type=jnp.float32)
        h = jnp.maximum(u * _dinv(d0_ref[...], d1_ref[...]) + bg_ref[...], 0.0)
        ol_ref[...] = h[:, :_W]
        oh_ref[...] = h[:, _W:]

    return pl.pallas_call(
        body,
        grid=(pl.cdiv(n, _R),),
        in_specs=[
            pl.BlockSpec((_R, _W), _blk),
            pl.BlockSpec((_R, _W), _blk),
            pl.BlockSpec((_R, _W), _blk),
            pl.BlockSpec((_W, dmid), _fix),
            pl.BlockSpec((1, dmid), _fix),
            pl.BlockSpec((_R, _W), _blk),
            pl.BlockSpec((_R, _W), _blk),
        ],
        out_specs=[pl.BlockSpec((_R, _W), _blk)] * 2,
        out_shape=[jax.ShapeDtypeStruct((n, _W), jnp.float32)] * 2,
    )(za, zb, hd, wg, bg, d0, d1)


def _tc_final(za, zb, y, bg, d0, d1, dout):
    n = y.shape[0]

    def body(za_ref, zb_ref, y_ref, bg_ref, d0_ref, d1_ref, o_ref):
        u = (za_ref[...] + zb_ref[...] + y_ref[...]) * _dinv(d0_ref[...], d1_ref[...])
        u = u[:, :dout] + bg_ref[...]
        m = jnp.max(u, axis=1, keepdims=True)
        lse = jnp.log(jnp.sum(jnp.exp(u - m), axis=1, keepdims=True))
        o_ref[...] = u - m - lse

    return pl.pallas_call(
        body,
        grid=(pl.cdiv(n, _R),),
        in_specs=[
            pl.BlockSpec((_R, _W), _blk),
            pl.BlockSpec((_R, _W), _blk),
            pl.BlockSpec((_R, _W), _blk),
            pl.BlockSpec((1, dout), _fix),
            pl.BlockSpec((_R, _W), _blk),
            pl.BlockSpec((_R, _W), _blk),
        ],
        out_specs=pl.BlockSpec((_R, dout), _blk),
        out_shape=jax.ShapeDtypeStruct((n, dout), jnp.float32),
    )(za, zb, y, bg, d0, d1)


def _pad_edges(src, dst, n):
    mult = 32 * 128 * 8
    e = src.shape[0]
    ep = (e + mult - 1) // mult * mult
    pad = ep - e
    if pad:
        fill_src = (jnp.arange(pad, dtype=jnp.int32) * 997) % n
        fill_dst = n + (jnp.arange(pad, dtype=jnp.int32) % 16)
        src = jnp.concatenate([src, fill_src])
        dst = jnp.concatenate([dst, fill_dst])
    return src, dst, ep


def _shard(a, ep, nway, bw=None):
    bw = _B if bw is None else bw
    return a.reshape(nway, ep // (nway * bw), bw)


def kernel(x_in, edge_index_in, ego_edge_index, W1, b1, Wg1, bg1,
           W2, b2, Wg2, bg2):
    n, d_feat = x_in.shape
    d_hid = Wg1.shape[1]
    d_out = Wg2.shape[1]
    inv_n = 1.0 / n

    ego_src = ego_edge_index[:, 1, :].reshape(-1)
    ego_dst = ego_edge_index[:, 0, :].reshape(-1)
    es, ed, ep_ego = _pad_edges(ego_src, ego_dst, n)
    gs, gd, ep_gcn = _pad_edges(edge_index_in[0], edge_index_in[1], n)

    b1r = b1.reshape(1, -1)
    b2r = b2.reshape(1, -1)
    bg1r = bg1.reshape(1, -1)
    bg2r = bg2.reshape(1, -1)

    deg0, deg1 = _make_deg(n, ep_gcn)(_shard(gd, ep_gcn, 32, 128))

    e1a, e1b = _make_seg_sum(n, ep_ego, 32)(
        x_in, x_in, _shard(es, ep_ego, 32), _shard(ed, ep_ego, 32))
    hd = _tc_l1(e1a, e1b, W1, b1r, deg0, deg1, inv_n)
    za, zb = _make_seg_sum(n, ep_gcn, 32)(
        hd, hd, _shard(gs, ep_gcn, 32), _shard(gd, ep_gcn, 32))
    h2_lo, h2_hi = _tc_mid(za, zb, hd, Wg1, bg1r, deg0, deg1)

    e2_lo, e2_hi = _make_seg_sum(n, ep_ego, 16)(
        h2_lo, h2_hi, _shard(es, ep_ego, 16), _shard(ed, ep_ego, 16))
    (y2p,) = _tc_layer(e2_lo, e2_hi, W2[:_W], W2[_W:], b2r, Wg2,
                       deg0, deg1, inv_n)
    z2a, z2b = _make_seg_sum(n, ep_gcn, 32)(
        y2p, y2p, _shard(gs, ep_gcn, 32), _shard(gd, ep_gcn, 32))
    return _tc_final(z2a, z2b, y2p, bg2r, deg0, deg1, d_out)


# speedup vs baseline: 1.2763x; 1.2763x over previous
import functools

import jax
import jax.numpy as jnp
from jax import lax
from jax.experimental import pallas as pl
from jax.experimental.pallas import tpu as pltpu
from jax.experimental.pallas import tpu_sc as plsc

_L = 16
_W = 128
_B = 64


def _fill_f32(ref, rows, cols, value):
    nv = cols // _L

    def body(k, carry):
        i = k // nv
        j = (k % nv) * _L
        ref[i, pl.ds(j, _L)] = jnp.full((_L,), value, jnp.float32)
        return carry

    lax.fori_loop(0, rows * nv, body, 0)


def _zero_shared(zbuf, acc, s, rows_per_tile, zrows):
    base = s * rows_per_tile
    off = 0
    while off < rows_per_tile:
        nr = min(zrows, rows_per_tile - off)
        pltpu.sync_copy(zbuf.at[pl.ds(0, nr)], acc.at[pl.ds(base + off, nr)])
        off += nr


def _copy_out(acc, out_ref, s, n_rows):
    rpt = (n_rows // 16) // 8 * 8
    pltpu.sync_copy(acc.at[pl.ds(s * rpt, rpt)], out_ref.at[pl.ds(s * rpt, rpt)])
    rem = n_rows - rpt * 16
    if rem:
        @pl.when(s == 15)
        def _():
            pltpu.sync_copy(acc.at[pl.ds(rpt * 16, rem)],
                            out_ref.at[pl.ds(rpt * 16, rem)])


def _acc_rows(n):
    return (n + 16 + 127) // 128 * 128


def _make_seg_sum(n, ep, nway):
    np_acc = _acc_rows(n)
    nb = ep // (nway * _B)
    nbc = min(nb, 80)
    assert nb % nbc == 0 and nbc % 2 == 0
    rows_per_tile_z = np_acc // 16
    mesh = plsc.VectorSubcoreMesh(core_axis_name="c", subcore_axis_name="s")

    @functools.partial(
        pl.kernel,
        out_type=(jax.ShapeDtypeStruct((n, _W), jnp.float32),
                  jax.ShapeDtypeStruct((n, _W), jnp.float32)),
        mesh=mesh,
        scratch_types=dict(
            idx_s=pltpu.VMEM((nbc, _B), jnp.int32),
            idx_d=pltpu.VMEM((nbc, _B), jnp.int32),
            rows_a=pltpu.VMEM((_B, _W), jnp.float32),
            rows_b=pltpu.VMEM((_B, _W), jnp.float32),
            sem_a=pltpu.SemaphoreType.DMA,
            sem_b=pltpu.SemaphoreType.DMA,
            acc=pltpu.VMEM_SHARED((np_acc, _W), jnp.float32),
        ),
    )
    def seg_sum(x0, x1, src3d, dst3d, out0, out1,
                idx_s, idx_d, rows_a, rows_b, sem_a, sem_b, acc):
        c = lax.axis_index("c")
        s = lax.axis_index("s")
        _fill_f32(rows_a, _B, _W, 0.0)
        _zero_shared(rows_a, acc, s, rows_per_tile_z, _B)
        slab = s if nway == 16 else s * 2 + c
        plsc.subcore_barrier()

        def run(x_ref, out_ref):
            def fire(rows, b, sem):
                pltpu.async_copy(x_ref.at[idx_s.at[b]], rows, sem)

            def drain(rows, b, sem):
                pltpu.make_async_copy(x_ref.at[idx_s.at[b]], rows, sem).wait()

            def scat(rows, b):
                pltpu.sync_copy(rows, acc.at[idx_d.at[b]], add=True)

            def chunk(ci, carry):
                r0 = pl.multiple_of(ci * nbc, 8)
                pltpu.sync_copy(src3d.at[slab, pl.ds(r0, nbc)], idx_s)
                pltpu.sync_copy(dst3d.at[slab, pl.ds(r0, nbc)], idx_d)
                fire(rows_a, 0, sem_a)

                def body(q, carry2):
                    b0 = 2 * q
                    fire(rows_b, b0 + 1, sem_b)
                    drain(rows_a, b0, sem_a)
                    scat(rows_a, b0)

                    @pl.when(b0 + 2 < nbc)
                    def _():
                        fire(rows_a, b0 + 2, sem_a)

                    drain(rows_b, b0 + 1, sem_b)
                    scat(rows_b, b0 + 1)
                    return carry2

                lax.fori_loop(0, nbc // 2, body, 0)
                return carry

            lax.fori_loop(0, nb // nbc, chunk, 0)
            plsc.subcore_barrier()
            _copy_out(acc, out_ref, s, n)

        @pl.when(c == 0)
        def _():
            run(x0, out0)

        @pl.when(c == 1)
        def _():
            run(x1, out1)

    return seg_sum


def _make_deg(n, ep):
    np_acc = _acc_rows(n)
    nb = ep // (32 * 128)
    rows_per_tile_z = np_acc // 16
    mesh = plsc.VectorSubcoreMesh(core_axis_name="c", subcore_axis_name="s")

    @functools.partial(
        pl.kernel,
        out_type=(jax.ShapeDtypeStruct((n, _L), jnp.float32),
                  jax.ShapeDtypeStruct((n, _L), jnp.float32)),
        mesh=mesh,
        scratch_types=dict(
            ones_r=pltpu.VMEM((128, _L), jnp.float32),
            idxb=pltpu.VMEM((nb, 128), jnp.int32),
            dacc=pltpu.VMEM_SHARED((np_acc, _L), jnp.float32),
        ),
    )
    def deg_kernel(dst3d, deg0, deg1, ones_r, idxb, dacc):
        c = lax.axis_index("c")
        s = lax.axis_index("s")
        _fill_f32(ones_r, 128, _L, 0.0)
        _zero_shared(ones_r, dacc, s, rows_per_tile_z, 128)
        _fill_f32(ones_r, 128, _L, 1.0)
        w = s * 2 + c
        pltpu.sync_copy(dst3d.at[w], idxb)
        plsc.subcore_barrier()

        def body(b, carry):
            pltpu.sync_copy(ones_r, dacc.at[idxb.at[b]], add=True)
            return carry

        lax.fori_loop(0, nb, body, 0)
        plsc.subcore_barrier()

        @pl.when(c == 0)
        def _():
            _copy_out(dacc, deg0, s, n)

        @pl.when(c == 1)
        def _():
            _copy_out(dacc, deg1, s, n)

    return deg_kernel


_R = 512


def _dinv(d0, d1):
    return lax.rsqrt(d0[:, :1] + d1[:, :1] + 1.0)


def _blk(i):
    return (i, 0)


def _fix(i):
    return (0, 0)


def _tc_l1(xa, xb, w, b, d0, d1, inv_n):
    n = xa.shape[0]
    dmid = w.shape[1]

    def body(xa_ref, xb_ref, w_ref, b_ref, d0_ref, d1_ref, o_ref):
        h = jnp.dot(xa_ref[...] + xb_ref[...], w_ref[...],
                    preferred_element_type=jnp.float32)
        h = jnp.maximum(h * inv_n + b_ref[...], 0.0)
        o_ref[...] = h * _dinv(d0_ref[...], d1_ref[...])

    return pl.pallas_call(
        body,
        grid=(pl.cdiv(n, _R),),
        in_specs=[
            pl.BlockSpec((_R, _W), _blk),
            pl.BlockSpec((_R, _W), _blk),
            pl.BlockSpec((_W, dmid), _fix),
            pl.BlockSpec((1, dmid), _fix),
            pl.BlockSpec((_R, _L), _blk),
            pl.BlockSpec((_R, _L), _blk),
        ],
        out_specs=pl.BlockSpec((_R, _W), _blk),
        out_shape=jax.ShapeDtypeStruct((n, _W), jnp.float32),
    )(xa, xb, w, b, d0, d1)


def _tc_layer(xa, xb, wa, wb, b, wg, d0, d1, inv_n):
    n = xa.shape[0]
    dmid = wa.shape[1]
    dout = wg.shape[1]
    nout = (dout + _W - 1) // _W

    def body(xa_ref, xb_ref, wa_ref, wb_ref, b_ref, wg_ref, d0_ref, d1_ref,
             *o_refs):
        h = (jnp.dot(xa_ref[...], wa_ref[...], preferred_element_type=jnp.float32)
             + jnp.dot(xb_ref[...], wb_ref[...], preferred_element_type=jnp.float32))
        h = jnp.maximum(h * inv_n + b_ref[...], 0.0)
        y = jnp.dot(h, wg_ref[...], preferred_element_type=jnp.float32)
        y = y * _dinv(d0_ref[...], d1_ref[...])
        for k in range(nout):
            lo = k * _W
            wk = min(_W, dout - lo)
            blk = y[:, lo:lo + wk]
            if wk < _W:
                blk = jnp.concatenate(
                    [blk, jnp.zeros((blk.shape[0], _W - wk), jnp.float32)],
                    axis=1)
            o_refs[k][...] = blk

    return pl.pallas_call(
        body,
        grid=(pl.cdiv(n, _R),),
        in_specs=[
            pl.BlockSpec((_R, _W), _blk),
            pl.BlockSpec((_R, _W), _blk),
            pl.BlockSpec((_W, dmid), _fix),
            pl.BlockSpec((_W, dmid), _fix),
            pl.BlockSpec((1, dmid), _fix),
            pl.BlockSpec((dmid, dout), _fix),
            pl.BlockSpec((_R, _L), _blk),
            pl.BlockSpec((_R, _L), _blk),
        ],
        out_specs=[pl.BlockSpec((_R, _W), _blk)] * nout,
        out_shape=[jax.ShapeDtypeStruct((n, _W), jnp.float32)] * nout,
    )(xa, xb, wa, wb, b, wg, d0, d1)


def _tc_mid(za, zb, hd, wg, bg, d0, d1):
    n = za.shape[0]
    dmid = wg.shape[1]

    def body(za_ref, zb_ref, hd_ref, wg_ref, bg_ref, d0_ref, d1_ref,
             ol_ref, oh_ref):
        u = jnp.dot(za_ref[...] + zb_ref[...] + hd_ref[...], wg_ref[...],
                    preferred_element_type=jnp.float32)
        h = jnp.maximum(u * _dinv(d0_ref[...], d1_ref[...]) + bg_ref[...], 0.0)
        ol_ref[...] = h[:, :_W]
        oh_ref[...] = h[:, _W:]

    return pl.pallas_call(
        body,
        grid=(pl.cdiv(n, _R),),
        in_specs=[
            pl.BlockSpec((_R, _W), _blk),
            pl.BlockSpec((_R, _W), _blk),
            pl.BlockSpec((_R, _W), _blk),
            pl.BlockSpec((_W, dmid), _fix),
            pl.BlockSpec((1, dmid), _fix),
            pl.BlockSpec((_R, _L), _blk),
            pl.BlockSpec((_R, _L), _blk),
        ],
        out_specs=[pl.BlockSpec((_R, _W), _blk)] * 2,
        out_shape=[jax.ShapeDtypeStruct((n, _W), jnp.float32)] * 2,
    )(za, zb, hd, wg, bg, d0, d1)


def _tc_final(za, zb, y, bg, d0, d1, dout):
    n = y.shape[0]

    def body(za_ref, zb_ref, y_ref, bg_ref, d0_ref, d1_ref, o_ref):
        u = (za_ref[...] + zb_ref[...] + y_ref[...]) * _dinv(d0_ref[...], d1_ref[...])
        u = u[:, :dout] + bg_ref[...]
        m = jnp.max(u, axis=1, keepdims=True)
        lse = jnp.log(jnp.sum(jnp.exp(u - m), axis=1, keepdims=True))
        o_ref[...] = u - m - lse

    return pl.pallas_call(
        body,
        grid=(pl.cdiv(n, _R),),
        in_specs=[
            pl.BlockSpec((_R, _W), _blk),
            pl.BlockSpec((_R, _W), _blk),
            pl.BlockSpec((_R, _W), _blk),
            pl.BlockSpec((1, dout), _fix),
            pl.BlockSpec((_R, _L), _blk),
            pl.BlockSpec((_R, _L), _blk),
        ],
        out_specs=pl.BlockSpec((_R, dout), _blk),
        out_shape=jax.ShapeDtypeStruct((n, dout), jnp.float32),
    )(za, zb, y, bg, d0, d1)


def _pad_edges(src, dst, n):
    mult = 32 * 128 * 8
    e = src.shape[0]
    ep = (e + mult - 1) // mult * mult
    pad = ep - e
    if pad:
        fill_src = (jnp.arange(pad, dtype=jnp.int32) * 997) % n
        fill_dst = n + (jnp.arange(pad, dtype=jnp.int32) % 16)
        src = jnp.concatenate([src, fill_src])
        dst = jnp.concatenate([dst, fill_dst])
    return src, dst, ep


def _shard(a, ep, nway, bw=None):
    bw = _B if bw is None else bw
    return a.reshape(nway, ep // (nway * bw), bw)


def kernel(x_in, edge_index_in, ego_edge_index, W1, b1, Wg1, bg1,
           W2, b2, Wg2, bg2):
    n, d_feat = x_in.shape
    d_hid = Wg1.shape[1]
    d_out = Wg2.shape[1]
    inv_n = 1.0 / n

    ego_src = ego_edge_index[:, 1, :].reshape(-1)
    ego_dst = ego_edge_index[:, 0, :].reshape(-1)
    es, ed, ep_ego = _pad_edges(ego_src, ego_dst, n)
    gs, gd, ep_gcn = _pad_edges(edge_index_in[0], edge_index_in[1], n)

    b1r = b1.reshape(1, -1)
    b2r = b2.reshape(1, -1)
    bg1r = bg1.reshape(1, -1)
    bg2r = bg2.reshape(1, -1)

    deg0, deg1 = _make_deg(n, ep_gcn)(_shard(gd, ep_gcn, 32, 128))

    e1a, e1b = _make_seg_sum(n, ep_ego, 32)(
        x_in, x_in, _shard(es, ep_ego, 32), _shard(ed, ep_ego, 32))
    hd = _tc_l1(e1a, e1b, W1, b1r, deg0, deg1, inv_n)
    za, zb = _make_seg_sum(n, ep_gcn, 32)(
        hd, hd, _shard(gs, ep_gcn, 32), _shard(gd, ep_gcn, 32))
    h2_lo, h2_hi = _tc_mid(za, zb, hd, Wg1, bg1r, deg0, deg1)

    e2_lo, e2_hi = _make_seg_sum(n, ep_ego, 16)(
        h2_lo, h2_hi, _shard(es, ep_ego, 16), _shard(ed, ep_ego, 16))
    (y2p,) = _tc_layer(e2_lo, e2_hi, W2[:_W], W2[_W:], b2r, Wg2,
                       deg0, deg1, inv_n)
    z2a, z2b = _make_seg_sum(n, ep_gcn, 32)(
        y2p, y2p, _shard(gs, ep_gcn, 32), _shard(gd, ep_gcn, 32))
    return _tc_final(z2a, z2b, y2p, bg2r, deg0, deg1, d_out)
